# Initial kernel scaffold; baseline (speedup 1.0000x reference)
import jax
import jax.numpy as jnp
from jax.experimental import pallas as pl


def kernel(x, edge_index, slot2_unused, batch, params):
    raise NotImplementedError("write your pallas kernel here")


# speedup vs baseline: 1.0419x; 1.0419x over previous
import functools

import jax
import jax.numpy as jnp
from jax import lax
from jax.experimental import pallas as pl
from jax.experimental.pallas import tpu as pltpu

N = 10000
E = 320000
FIN = 128
H = 128
NG = 128


def _mm_body(x_ref, w_ref, o_ref):
    o_ref[...] = jnp.dot(x_ref[...], w_ref[...],
                         preferred_element_type=jnp.float32)


def _matmul(x, w):
    m, k = x.shape
    k2, n = w.shape
    return pl.pallas_call(
        _mm_body,
        out_shape=jax.ShapeDtypeStruct((m, n), jnp.float32),
    )(x, w)


def _greedy_match(score, src, dst, node_mask):
    n = node_mask.shape[0]
    order = jnp.argsort(-score, stable=True)
    cluster0 = jnp.full(n, -1, jnp.int32)
    eoc0 = jnp.full(n, -1, jnp.int32)

    def step(carry, i):
        cluster, eoc, k = carry
        s = src[i]
        d = dst[i]
        cond = (s != d) & (cluster[s] < 0) & (cluster[d] < 0)
        ks = jnp.where(cond, k, cluster[s])
        kd = jnp.where(cond, k, cluster[d])
        cluster = cluster.at[s].set(ks)
        cluster = cluster.at[d].set(kd)
        eoc = eoc.at[k].set(jnp.where(cond, i.astype(jnp.int32), eoc[k]))
        k = k + cond.astype(jnp.int32)
        return (cluster, eoc, k), None

    (cluster, eoc, k), _ = lax.scan(step, (cluster0, eoc0, jnp.int32(0)), order)
    unmatched = node_mask & (cluster < 0)
    ranks = jnp.cumsum(unmatched.astype(jnp.int32)) - 1
    cluster = jnp.where(unmatched, k + ranks, cluster)
    K = k + unmatched.astype(jnp.int32).sum()
    return cluster, eoc, K


def _gcn_conv(x, src, dst, W, b, ew=None):
    n = x.shape[0]
    h = _matmul(x, W)
    sl = jnp.arange(n, dtype=src.dtype)
    s = jnp.concatenate([src, sl])
    d = jnp.concatenate([dst, sl])
    if ew is None:
        w = jnp.ones(s.shape[0], x.dtype)
    else:
        w = jnp.concatenate([ew, jnp.ones(n, x.dtype)])
    deg = jnp.zeros(n, x.dtype).at[d].add(w)
    dis = 1.0 / jnp.sqrt(deg)
    coef = (w * dis[s] * dis[d])[:, None]
    return jnp.zeros_like(h).at[d].add(coef * h[s]) + b


def _bn(x, g, b, mask=None, count=None):
    if mask is None:
        m = x.mean(0)
        v = x.var(0)
        xc = x - m
    else:
        mf = mask[:, None].astype(x.dtype)
        cf = count.astype(x.dtype)
        m = (x * mf).sum(0) / cf
        xc = x - m
        v = (xc * xc * mf).sum(0) / cf
    return xc / jnp.sqrt(v + 1e-5) * g + b


def _edge_scores(x, src, dst, pw, pb, emask=None):
    n = x.shape[0]
    e = jnp.concatenate([x[src], x[dst]], axis=1) @ pw + pb
    if emask is not None:
        e = jnp.where(emask, e, -jnp.inf)
    emax = jnp.full(n, -jnp.inf, x.dtype).at[dst].max(e)
    ee = jnp.exp(e - emax[dst])
    if emask is not None:
        ee = jnp.where(emask, ee, 0.0)
    den = jnp.zeros(n, x.dtype).at[dst].add(ee)
    if emask is None:
        return ee / den[dst] + 0.5
    return jnp.where(emask, ee / den[dst], 0.0) + 0.5


def _pool_structs(score, src, dst, batch, node_mask, edge_mask):
    n = node_mask.shape[0]
    m = edge_mask.shape[0]
    cl, eoc, K = _greedy_match(score, src, dst, node_mask)
    cls = jnp.where(node_mask, cl, 0)
    ns = cls[src]
    nd = cls[dst]
    valid = edge_mask & (ns != nd)
    sent = jnp.int32(n * n)
    ids = jnp.where(valid, ns * n + nd, sent)
    sids = jnp.sort(ids)
    um = (sids < sent) & jnp.concatenate([jnp.ones(1, bool), sids[1:] != sids[:-1]])
    pos = jnp.where(um, jnp.cumsum(um.astype(jnp.int32)) - 1, m)
    uids = jnp.full(m, sent, jnp.int32).at[pos].set(sids, mode='drop')
    ne_mask = uids < sent
    nsrc = jnp.where(ne_mask, uids // n, 0).astype(jnp.int32)
    ndst = jnp.where(ne_mask, uids % n, 0).astype(jnp.int32)
    idxs = jnp.where(node_mask, jnp.arange(n, dtype=jnp.int32), -1)
    maxnode = jnp.full(n, -1, jnp.int32).at[cls].max(idxs)
    nb = jnp.where(maxnode >= 0, batch[jnp.maximum(maxnode, 0)], 0).astype(batch.dtype)
    return (cls, eoc, nsrc, ndst, nb, K, ne_mask)


def _apply_pool(x, score, st, node_mask):
    cl, eoc, nsrc, ndst, nb, K, ne_mask = st
    per = jnp.where(eoc >= 0, score[jnp.maximum(eoc, 0)], 1.0)
    xm = x * node_mask[:, None].astype(x.dtype)
    nx = jnp.zeros((x.shape[0], x.shape[1]), x.dtype).at[cl].add(xm) * per[:, None]
    nmask = jnp.arange(x.shape[0]) < K
    return nx, nsrc, ndst, nb, nmask, ne_mask


def kernel(x, edge_index, slot2_unused, batch, params):
    src = edge_index[:, 0]
    dst = edge_index[:, 1]
    n = x.shape[0]
    nmask1 = jnp.ones(n, bool)
    emask1 = jnp.ones(src.shape[0], bool)
    h1 = jax.nn.relu(_bn(_gcn_conv(x, src, dst, params['W1'], params['b1']),
                         params['g1'], params['be1']))
    h1 = jax.nn.relu(_bn(_gcn_conv(h1, src, dst, params['W2'], params['b2']),
                         params['g2'], params['be2']))
    s1 = _edge_scores(h1, src, dst, params['pw1'], params['pb1'])
    p1 = _pool_structs(s1, src, dst, batch, nmask1, emask1)
    h2, src2, dst2, batch2, nmask2, emask2 = _apply_pool(h1, s1, p1, nmask1)
    K1 = p1[5]
    ew2 = emask2.astype(x.dtype)
    h2 = jax.nn.relu(_bn(_gcn_conv(h2, src2, dst2, params['W3'], params['b3'], ew2),
                         params['g3'], params['be3'], nmask2, K1))
    h2 = jax.nn.relu(_bn(_gcn_conv(h2, src2, dst2, params['W4'], params['b4'], ew2),
                         params['g4'], params['be4'], nmask2, K1))
    s2 = _edge_scores(h2, src2, dst2, params['pw2'], params['pb2'], emask2)
    p2 = _pool_structs(s2, src2, dst2, batch2, nmask2, emask2)
    h3, src3, dst3, batch3, nmask3, emask3 = _apply_pool(h2, s2, p2, nmask2)
    K2 = p2[5]
    ew3 = emask3.astype(x.dtype)
    h3 = jax.nn.relu(_bn(_gcn_conv(h3, src3, dst3, params['W5'], params['b5'], ew3),
                         params['g5'], params['be5'], nmask3, K2))
    mf3 = nmask3.astype(x.dtype)
    cnt = jnp.zeros(NG, x.dtype).at[batch3].add(mf3)
    sums = jnp.zeros((NG, h3.shape[1]), x.dtype).at[batch3].add(h3 * mf3[:, None])
    g = sums / jnp.maximum(cnt, 1.0)[:, None]
    f = jax.nn.relu(g @ params['fw1'] + params['fb1'])
    out = jax.nn.sigmoid(f @ params['fw2'] + params['fb2'])
    return out.reshape(-1)
